# Initial kernel scaffold; baseline (speedup 1.0000x reference)
import jax
import jax.numpy as jnp
from jax.experimental import pallas as pl


def kernel(feat_x, index_pos, index_neg):
    raise NotImplementedError("write your pallas kernel here")


# speedup vs baseline: 75.5631x; 75.5631x over previous
import jax
import jax.numpy as jnp
from jax.experimental import pallas as pl
from jax.experimental.pallas import tpu as pltpu

_EPS = 1e-8


def _body(a_ref, h_ref, dinv_ref, t_ref, out_ref):
    i = pl.program_id(0)
    t = t_ref[0]
    a = a_ref[0]
    h = h_ref[0]
    dinv = dinv_ref[0]

    rows = jax.lax.broadcasted_iota(jnp.int32, (40, 2048), 1)
    m = ((rows >= a) & (rows < h)).astype(jnp.float32)
    seg = jnp.dot(m, t, preferred_element_type=jnp.float32) * dinv

    jidx = jax.lax.broadcasted_iota(jnp.int32, (40, 1), 0)
    posmask = (jidx < 20).astype(jnp.float32)
    center = jnp.sum(seg * posmask, axis=0, keepdims=True) / 20.0

    dot = jnp.sum(seg * center, axis=1, keepdims=True)
    na = jnp.maximum(jnp.sqrt(jnp.sum(seg * seg, axis=1, keepdims=True)), _EPS)
    nc = jnp.maximum(jnp.sqrt(jnp.sum(center * center, axis=1, keepdims=True)), _EPS)
    one_minus_cos = 1.0 - dot / (na * nc)

    l1 = jnp.sum(one_minus_cos * posmask) / 20.0
    l2 = jnp.sum(jnp.exp(-one_minus_cos) * (1.0 - posmask)) / 20.0
    contrib = jnp.reshape((l1 + l2) * 0.125, (1, 1))

    @pl.when(i == 0)
    def _():
        out_ref[...] = jnp.zeros((1, 1), jnp.float32)

    out_ref[...] += contrib


def kernel(feat_x, index_pos, index_neg):
    ip = index_pos.astype(jnp.int32)
    ineg = index_neg.astype(jnp.int32)
    a = jnp.concatenate([ip[:, 0::2], ineg[:, 0::2]], axis=1)
    b = jnp.concatenate([ip[:, 1::2], ineg[:, 1::2]], axis=1)
    h = jnp.maximum(b, a + 1)
    dinv = 1.0 / jnp.maximum(b - a, 1).astype(jnp.float32)

    af = a[..., None]
    hf = h[..., None]
    dinvf = dinv[..., None]

    out = pl.pallas_call(
        _body,
        grid=(feat_x.shape[0],),
        in_specs=[
            pl.BlockSpec((1, 40, 1), lambda i: (i, 0, 0)),
            pl.BlockSpec((1, 40, 1), lambda i: (i, 0, 0)),
            pl.BlockSpec((1, 40, 1), lambda i: (i, 0, 0)),
            pl.BlockSpec((1, 2048, 512), lambda i: (i, 0, 0)),
        ],
        out_specs=pl.BlockSpec((1, 1), lambda i: (0, 0)),
        out_shape=jax.ShapeDtypeStruct((1, 1), jnp.float32),
    )(af, hf, dinvf, feat_x)
    return out[0, 0]
